# baseline (speedup 1.0000x reference)
import jax
import jax.numpy as jnp
import numpy as np
from jax.experimental import pallas as pl
from jax.experimental.pallas import tpu as pltpu

_NUM = 256
_NPATCH = 256
_QDIM = 256
_KDIM = 256
_FB = 8
_INDIM = 48
_IMG = 64
_SCALE = 1.0 / (48.0 ** 0.5)
_BB = 8
_HB = 8

_PJ = np.zeros((192, 192), np.float32)
for _c in range(3):
    for _w1 in range(4):
        for _w0 in range(16):
            _PJ[64 * _c + 16 * _w1 + _w0, 48 * _w1 + 3 * _w0 + _c] = 1.0
_PPI = np.zeros((256, 256), np.float32)
for _p in range(256):
    _PPI[_p, 64 * (_p % 4) + _p // 4] = 1.0


def _transposer(xin_ref, pj_ref, out_ref):
    for hh in range(_HB):
        out_ref[hh] = jax.lax.dot_general(
            xin_ref[hh], pj_ref[...], (((0,), (0,)), ((), ())),
            preferred_element_type=jnp.float32)


def _stage_a(rp_ref, ppi_ref, wqt_ref, bq_ref, wkt_ref, bk_ref, mcolt_ref,
             pa_ref, cm_ref):
    for i in range(_BB):
        x = rp_ref[:, i, :]
        rp_pi = jnp.concatenate(
            [x[:, 48 * w1:48 * w1 + 48] for w1 in range(4)], axis=0)
        rp = jnp.dot(ppi_ref[...], rp_pi,
                     preferred_element_type=jnp.float32)
        q = jnp.dot(rp, wqt_ref[...], preferred_element_type=jnp.float32) + bq_ref[...]
        k = jnp.dot(rp, wkt_ref[...], preferred_element_type=jnp.float32) + bk_ref[...]
        att = jnp.dot(q, k, preferred_element_type=jnp.float32) * _SCALE
        att = att - jnp.max(att, axis=1, keepdims=True)
        e = jnp.exp(att)
        e = e / jnp.sum(e, axis=1, keepdims=True)
        pa_ref[i, 0, :] = jnp.sum(e, axis=0)
        cm_ref[:, i, 0, :] = jax.lax.dot_general(
            mcolt_ref[...], rp, (((1,), (1,)), ((), ())),
            preferred_element_type=jnp.float32)


def _stage_b(pa_ref, cm_ref, w1t_ref, b1_ref, w2t_ref, b2_ref,
             act_ref, sel_ref):
    pa = pa_ref[:, 0, :]
    iota = jax.lax.broadcasted_iota(jnp.int32, (_NUM, _NPATCH), 1)
    cols = []
    for _ in range(_FB):
        m = jnp.max(pa, axis=1, keepdims=True)
        eq = pa >= m
        idx = jnp.min(jnp.where(eq, iota, _NPATCH), axis=1, keepdims=True)
        sel = iota == idx
        pa = jnp.where(sel, -1.0, pa)
        row = idx // _IMG
        col = idx - row * _IMG
        cx = (row.astype(jnp.float32) + 2.0) * (1.0 / _IMG)
        cy = (col.astype(jnp.float32) + 2.0) * (1.0 / _IMG)
        r = jnp.sum(jnp.where(sel, cm_ref[0, :, 0, :], 0.0), axis=1, keepdims=True)
        g = jnp.sum(jnp.where(sel, cm_ref[1, :, 0, :], 0.0), axis=1, keepdims=True)
        b = jnp.sum(jnp.where(sel, cm_ref[2, :, 0, :], 0.0), axis=1, keepdims=True)
        cols.extend([cx, cy, r, g, b])
    feats = jnp.concatenate(cols, axis=1)
    h = jnp.dot(feats, w1t_ref[...], preferred_element_type=jnp.float32) + b1_ref[...]
    logits = jnp.dot(h, w2t_ref[...], preferred_element_type=jnp.float32) + b2_ref[...]
    lm = jnp.max(logits, axis=1, keepdims=True)
    e = jnp.exp(logits - lm)
    act_ref[...] = e / jnp.sum(e, axis=1, keepdims=True)
    li = jax.lax.broadcasted_iota(jnp.int32, logits.shape, 1)
    sel_idx = jnp.min(jnp.where(logits >= lm, li, logits.shape[1]), axis=1)
    sel_ref[0, :] = sel_idx


def kernel(input, Wq, bq, Wk, bk, W1, b1, W2, b2):
    xin = input.transpose(1, 3, 2, 0).reshape(_IMG, 192, _NUM)
    mcolt = np.zeros((8, _INDIM), np.float32)
    for c in range(3):
        mcolt[c, np.arange(16) * 3 + c] = 1.0 / (16.0 * 255.0)
    mcolt = jnp.asarray(mcolt)

    rp3 = pl.pallas_call(
        _transposer,
        grid=(_IMG // _HB,),
        in_specs=[
            pl.BlockSpec((_HB, 192, _NUM), lambda h: (h, 0, 0)),
            pl.BlockSpec((192, 192), lambda h: (0, 0)),
        ],
        out_specs=pl.BlockSpec((_HB, _NUM, 192), lambda h: (h, 0, 0)),
        out_shape=jax.ShapeDtypeStruct((_IMG, _NUM, 192), jnp.float32),
    )(xin, jnp.asarray(_PJ))

    pa, cm = pl.pallas_call(
        _stage_a,
        grid=(_NUM // _BB,),
        in_specs=[
            pl.BlockSpec((_IMG, _BB, 192), lambda b: (0, b, 0)),
            pl.BlockSpec((_NPATCH, _NPATCH), lambda b: (0, 0)),
            pl.BlockSpec((_INDIM, _QDIM), lambda b: (0, 0)),
            pl.BlockSpec((1, _QDIM), lambda b: (0, 0)),
            pl.BlockSpec((_INDIM, _KDIM), lambda b: (0, 0)),
            pl.BlockSpec((1, _KDIM), lambda b: (0, 0)),
            pl.BlockSpec((8, _INDIM), lambda b: (0, 0)),
        ],
        out_specs=[
            pl.BlockSpec((_BB, 1, _NPATCH), lambda b: (b, 0, 0)),
            pl.BlockSpec((8, _BB, 1, _NPATCH), lambda b: (0, b, 0, 0)),
        ],
        out_shape=[
            jax.ShapeDtypeStruct((_NUM, 1, _NPATCH), jnp.float32),
            jax.ShapeDtypeStruct((8, _NUM, 1, _NPATCH), jnp.float32),
        ],
    )(rp3, jnp.asarray(_PPI), Wq.T, bq.reshape(1, -1), Wk.T,
      bk.reshape(1, -1), mcolt)

    actions, selected = pl.pallas_call(
        _stage_b,
        out_shape=[
            jax.ShapeDtypeStruct((_NUM, 15), jnp.float32),
            jax.ShapeDtypeStruct((1, _NUM), jnp.int32),
        ],
    )(pa, cm, W1.T, b1.reshape(1, -1), W2.T, b2.reshape(1, -1))

    return selected.reshape(_NUM), actions
